# baseline (speedup 1.0000x reference)
import functools

import jax
import jax.numpy as jnp
from jax import lax
from jax.experimental import pallas as pl
from jax.experimental.pallas import tpu as pltpu
from jax.experimental.pallas import tpu_sc as plsc

NU = 25000
D = 64
DH = D // 2
E = 400000
B = 4096
NC = 2
NS = 16
RT = 1568
NPAD = NS * RT
JUNK = NU
CH = 128
CPT = 200
EPAD = CPT * CH * NS
SB = 20
NSUP = CPT // SB
NBUF = 2
SCCH = 112
NSC = RT // SCCH
PPT = B // (NC * NS)

F32 = jnp.float32
I32 = jnp.int32

MESH = plsc.VectorSubcoreMesh(
    core_axis_name="c", subcore_axis_name="s", num_cores=NC, num_subcores=NS
)
CPARAMS = pltpu.CompilerParams(use_tc_tiling_on_sc=False, needs_layout_passes=False)


def _rsqrt_newton(x):
    i = lax.bitcast_convert_type(x, I32)
    i = 0x5F3759DF - jnp.right_shift(i, 1)
    y = lax.bitcast_convert_type(i, F32)
    for _ in range(3):
        y = y * (1.5 - 0.5 * x * y * y)
    return y


@functools.partial(
    pl.kernel,
    out_type=(
        jax.ShapeDtypeStruct((NC * NPAD,), F32),
        jax.ShapeDtypeStruct((NC * NPAD, DH), F32),
        jax.ShapeDtypeStruct((NC * NPAD, DH), F32),
    ),
    mesh=MESH,
    compiler_params=CPARAMS,
    scratch_types=[
        pltpu.VMEM_SHARED((NPAD,), F32),
        pltpu.VMEM((CPT, CH), I32),
        pltpu.VMEM((RT,), F32),
        pltpu.VMEM((RT,), F32),
        pltpu.VMEM((CH,), F32),
        pltpu.VMEM((SCCH, D), F32),
        pltpu.VMEM((SCCH, DH), F32),
        pltpu.VMEM((SCCH, DH), F32),
    ],
)
def _k_deg(rows_hbm, e0_hbm, s_hbm, f0lo_hbm, f0hi_hbm,
           deg_sp, idxr, zbuf, sbuf, ones, fbuf, fblo, fbhi):
    cid = lax.axis_index("c")
    sid = lax.axis_index("s")
    rbase = sid * RT

    def fz(i, _):
        zbuf[pl.ds(i * 16, 16)] = jnp.zeros((16,), F32)
        return 0

    lax.fori_loop(0, RT // 16, fz, 0)
    for i in range(CH // 16):
        ones[pl.ds(i * 16, 16)] = jnp.ones((16,), F32)
    pltpu.sync_copy(zbuf, deg_sp.at[pl.ds(rbase, RT)])
    pltpu.sync_copy(rows_hbm.at[cid, pl.ds(sid * CPT, CPT)], idxr)
    plsc.subcore_barrier()

    def deg_add(j, _):
        pltpu.sync_copy(ones, deg_sp.at[idxr.at[j]], add=True)
        return 0

    lax.fori_loop(0, CPT, deg_add, 0)
    plsc.subcore_barrier()

    pltpu.sync_copy(deg_sp.at[pl.ds(rbase, RT)], zbuf)

    def newton(i, _):
        x = jnp.maximum(zbuf[pl.ds(i * 16, 16)], 1.0)
        sbuf[pl.ds(i * 16, 16)] = _rsqrt_newton(x)
        return 0

    lax.fori_loop(0, RT // 16, newton, 0)
    pltpu.sync_copy(sbuf, s_hbm.at[pl.ds(cid * NPAD + rbase, RT)])

    jbase = cid * NPAD + rbase

    def f0_chunk(c, _):
        pltpu.sync_copy(e0_hbm.at[pl.ds(jbase + c * SCCH, SCCH), :], fbuf)

        def grp(g, _):
            sv16 = sbuf[pl.ds(c * SCCH + g * 16, 16)]
            for r16 in range(16):
                r = g * 16 + r16
                sv = sv16[r16]
                for d in range(DH // 16):
                    sl = pl.ds(d * 16, 16)
                    fblo[r, sl] = fbuf[r, sl] * sv
                    fbhi[r, sl] = fbuf[r, pl.ds(DH + d * 16, 16)] * sv
            return 0

        lax.fori_loop(0, SCCH // 16, grp, 0)
        pltpu.sync_copy(fblo, f0lo_hbm.at[pl.ds(jbase + c * SCCH, SCCH), :])
        pltpu.sync_copy(fbhi, f0hi_hbm.at[pl.ds(jbase + c * SCCH, SCCH), :])
        return 0

    lax.fori_loop(0, NSC, f0_chunk, 0)


@functools.partial(
    pl.kernel,
    out_type=(
        jax.ShapeDtypeStruct((NC * NPAD, DH), F32),
        jax.ShapeDtypeStruct((NC * NPAD, DH), F32),
    ),
    mesh=MESH,
    compiler_params=CPARAMS,
    scratch_types=[
        pltpu.VMEM_SHARED((NPAD, DH), F32),
        pltpu.VMEM_SHARED((NPAD, DH), F32),
        pltpu.VMEM((SB, CH), I32),
        pltpu.VMEM((SB, CH), I32),
        pltpu.VMEM((CH, DH), F32),
        pltpu.VMEM((CH, DH), F32),
        pltpu.VMEM((RT,), F32),
        pltpu.SemaphoreType.DMA,
        pltpu.SemaphoreType.DMA,
    ],
)
def _k_layer(rows_hbm, cols_hbm, flo_in, fhi_in, s_hbm, flo_out, fhi_out,
             src_sp, g_sp, idxr, idxc, m0, m1, sbuf, s0, s1):
    cid = lax.axis_index("c")
    sid = lax.axis_index("s")
    rbase = sid * RT
    cbase = sid * CPT
    ms = (m0, m1)
    sems = (s0, s1)
    jbase = cid * NPAD + rbase
    obase = (1 - cid) * NPAD + rbase

    pltpu.sync_copy(s_hbm.at[pl.ds(jbase, RT)], sbuf)

    for h, (f_in, f_out) in enumerate(((flo_in, flo_out), (fhi_in, fhi_out))):
        pltpu.sync_copy(f_in.at[pl.ds(obase, RT), :],
                        src_sp.at[pl.ds(rbase, RT), :])

        def fz(r, _):
            for d in range(DH // 16):
                m0[r, pl.ds(d * 16, 16)] = jnp.zeros((16,), F32)
            return 0

        lax.fori_loop(0, SCCH, fz, 0)

        def zc(c, _):
            pltpu.sync_copy(m0.at[pl.ds(0, SCCH), :],
                            g_sp.at[pl.ds(rbase + c * SCCH, SCCH), :])
            return 0

        lax.fori_loop(0, NSC, zc, 0)
        plsc.subcore_barrier()

        for u in range(NSUP):
            off = cbase + u * SB
            pltpu.sync_copy(rows_hbm.at[cid, pl.ds(off, SB)], idxr)
            pltpu.sync_copy(cols_hbm.at[cid, pl.ds(off, SB)], idxc)

            def edge_loop(t, _):
                descs = []
                for b in range(NBUF):
                    j = t * NBUF + b
                    descs.append(
                        pltpu.async_copy(src_sp.at[idxc.at[j]], ms[b], sems[b]))
                for b in range(NBUF):
                    j = t * NBUF + b
                    descs[b].wait()
                    pltpu.sync_copy(ms[b], g_sp.at[idxr.at[j]], add=True)
                return 0

            lax.fori_loop(0, SB // NBUF, edge_loop, 0)
        plsc.subcore_barrier()

        def scale_chunk(c, _):
            pltpu.sync_copy(g_sp.at[pl.ds(rbase + c * SCCH, SCCH), :],
                            m0.at[pl.ds(0, SCCH), :])

            def grp(g, _):
                sv16 = sbuf[pl.ds(c * SCCH + g * 16, 16)]
                dv16 = sv16 * sv16
                for r16 in range(16):
                    r = g * 16 + r16
                    dv = dv16[r16]
                    for d in range(DH // 16):
                        sl = pl.ds(d * 16, 16)
                        m0[r, sl] = m0[r, sl] * dv
                return 0

            lax.fori_loop(0, SCCH // 16, grp, 0)
            pltpu.sync_copy(m0.at[pl.ds(0, SCCH), :],
                            f_out.at[pl.ds(jbase + c * SCCH, SCCH), :])
            return 0

        lax.fori_loop(0, NSC, scale_chunk, 0)
        plsc.subcore_barrier()


@functools.partial(
    pl.kernel,
    out_type=jax.ShapeDtypeStruct((B,), F32),
    mesh=MESH,
    compiler_params=CPARAMS,
    scratch_types=[
        pltpu.VMEM((PPT,), I32),
        pltpu.VMEM((PPT,), I32),
        pltpu.VMEM((PPT, DH), F32),
        pltpu.VMEM((PPT, DH), F32),
        pltpu.VMEM((PPT, DH), F32),
        pltpu.VMEM((PPT, DH), F32),
        pltpu.VMEM((PPT,), F32),
        pltpu.VMEM((PPT,), F32),
        pltpu.VMEM((PPT,), F32),
    ],
)
def _k_decode(f0lo, f0hi, f1lo, f1hi, f2lo, f2hi, f3lo, f3hi,
              s_flat, uj_hbm, ij_hbm, gamma,
              uidx, iidx, bul, buh, bil, bih, su, si, gbuf):
    cid = lax.axis_index("c")
    sid = lax.axis_index("s")
    base = (cid * NS + sid) * PPT
    pltpu.sync_copy(uj_hbm.at[pl.ds(base, PPT)], uidx)
    pltpu.sync_copy(ij_hbm.at[pl.ds(base, PPT)], iidx)
    for tab, buf, idx in (
        (f0lo, bul, uidx), (f0hi, buh, uidx),
        (f0lo, bil, iidx), (f0hi, bih, iidx),
    ):
        pltpu.sync_copy(tab.at[idx], buf)
    for tab, buf, idx in (
        (f1lo, bul, uidx), (f2lo, bul, uidx), (f3lo, bul, uidx),
        (f1hi, buh, uidx), (f2hi, buh, uidx), (f3hi, buh, uidx),
        (f1lo, bil, iidx), (f2lo, bil, iidx), (f3lo, bil, iidx),
        (f1hi, bih, iidx), (f2hi, bih, iidx), (f3hi, bih, iidx),
    ):
        pltpu.sync_copy(tab.at[idx], buf, add=True)
    pltpu.sync_copy(s_flat.at[uidx], su)
    pltpu.sync_copy(s_flat.at[iidx], si)

    def grp(g, _):
        rows = lax.iota(I32, 16) + g * 16
        acc = jnp.zeros((16,), F32)
        for d in range(DH):
            cols = jnp.full((16,), d, I32)
            acc = acc + (plsc.load_gather(bul, [rows, cols])
                         * plsc.load_gather(bil, [rows, cols]))
            acc = acc + (plsc.load_gather(buh, [rows, cols])
                         * plsc.load_gather(bih, [rows, cols]))
        sl = pl.ds(g * 16, 16)
        gbuf[sl] = acc / (su[sl] * si[sl] * 16.0)
        return 0

    lax.fori_loop(0, PPT // 16, grp, 0)
    pltpu.sync_copy(gbuf, gamma.at[pl.ds(base, PPT)])


@jax.jit
def kernel(user_emb, item_emb, edge_index, users, items):
    src = edge_index[0].astype(I32)
    dst = edge_index[1].astype(I32)
    padr = jnp.full((EPAD - E,), JUNK, I32)
    padc = jnp.zeros((EPAD - E,), I32)
    rows3d = jnp.stack([
        jnp.concatenate([src, padr]),
        jnp.concatenate([dst, padr]),
    ]).reshape(NC, NS * CPT, CH)
    cols3d = jnp.stack([
        jnp.concatenate([dst, padc]),
        jnp.concatenate([src, padc]),
    ]).reshape(NC, NS * CPT, CH)
    zpad = jnp.zeros((NPAD - NU, D), F32)
    e0p = jnp.concatenate([user_emb, zpad, item_emb, zpad], axis=0)

    s1d, f0lo, f0hi = _k_deg(rows3d, e0p)
    f1lo, f1hi = _k_layer(rows3d, cols3d, f0lo, f0hi, s1d)
    f2lo, f2hi = _k_layer(rows3d, cols3d, f1lo, f1hi, s1d)
    f3lo, f3hi = _k_layer(rows3d, cols3d, f2lo, f2hi, s1d)
    gamma = _k_decode(f0lo, f0hi, f1lo, f1hi, f2lo, f2hi, f3lo, f3hi,
                      s1d, users.astype(I32), (items.astype(I32) + NPAD))
    return gamma


# speedup vs baseline: 1.2714x; 1.2714x over previous
import functools

import jax
import jax.numpy as jnp
from jax import lax
from jax.experimental import pallas as pl
from jax.experimental.pallas import tpu as pltpu
from jax.experimental.pallas import tpu_sc as plsc

NU = 25000
D = 64
DH = D // 2
E = 400000
B = 4096
NC = 2
NS = 16
RT = 1568
NPAD = NS * RT
JUNK = NU
CH = 128
CPT = 200
EPAD = CPT * CH * NS
SB = 40
NSUP = CPT // SB
NBUF = 4
SCCH = 112
NSC = RT // SCCH
PPT = B // (NC * NS)

F32 = jnp.float32
I32 = jnp.int32

MESH = plsc.VectorSubcoreMesh(
    core_axis_name="c", subcore_axis_name="s", num_cores=NC, num_subcores=NS
)
CPARAMS = pltpu.CompilerParams(use_tc_tiling_on_sc=False, needs_layout_passes=False)


def _rsqrt_newton(x):
    i = lax.bitcast_convert_type(x, I32)
    i = 0x5F3759DF - jnp.right_shift(i, 1)
    y = lax.bitcast_convert_type(i, F32)
    for _ in range(3):
        y = y * (1.5 - 0.5 * x * y * y)
    return y


@functools.partial(
    pl.kernel,
    out_type=(
        jax.ShapeDtypeStruct((NC * NPAD,), F32),
        jax.ShapeDtypeStruct((NC * NPAD, DH), F32),
        jax.ShapeDtypeStruct((NC * NPAD, DH), F32),
    ),
    mesh=MESH,
    compiler_params=CPARAMS,
    scratch_types=[
        pltpu.VMEM_SHARED((NPAD,), F32),
        pltpu.VMEM((CPT, CH), I32),
        pltpu.VMEM((RT,), F32),
        pltpu.VMEM((RT,), F32),
        pltpu.VMEM((CH,), F32),
        pltpu.VMEM((SCCH, D), F32),
        pltpu.VMEM((SCCH, DH), F32),
        pltpu.VMEM((SCCH, DH), F32),
    ],
)
def _k_deg(rows_hbm, e0_hbm, s_hbm, f0lo_hbm, f0hi_hbm,
           deg_sp, idxr, zbuf, sbuf, ones, fbuf, fblo, fbhi):
    cid = lax.axis_index("c")
    sid = lax.axis_index("s")
    rbase = sid * RT

    def fz(i, _):
        zbuf[pl.ds(i * 16, 16)] = jnp.zeros((16,), F32)
        return 0

    lax.fori_loop(0, RT // 16, fz, 0)
    for i in range(CH // 16):
        ones[pl.ds(i * 16, 16)] = jnp.ones((16,), F32)
    pltpu.sync_copy(zbuf, deg_sp.at[pl.ds(rbase, RT)])
    pltpu.sync_copy(rows_hbm.at[cid, pl.ds(sid * CPT, CPT)], idxr)
    plsc.subcore_barrier()

    def deg_add(j, _):
        pltpu.sync_copy(ones, deg_sp.at[idxr.at[j]], add=True)
        return 0

    lax.fori_loop(0, CPT, deg_add, 0)
    plsc.subcore_barrier()

    pltpu.sync_copy(deg_sp.at[pl.ds(rbase, RT)], zbuf)

    def newton(i, _):
        x = jnp.maximum(zbuf[pl.ds(i * 16, 16)], 1.0)
        sbuf[pl.ds(i * 16, 16)] = _rsqrt_newton(x)
        return 0

    lax.fori_loop(0, RT // 16, newton, 0)
    pltpu.sync_copy(sbuf, s_hbm.at[pl.ds(cid * NPAD + rbase, RT)])

    jbase = cid * NPAD + rbase

    def f0_chunk(c, _):
        pltpu.sync_copy(e0_hbm.at[pl.ds(jbase + c * SCCH, SCCH), :], fbuf)

        def grp(g, _):
            sv16 = sbuf[pl.ds(c * SCCH + g * 16, 16)]
            for r16 in range(16):
                r = g * 16 + r16
                sv = sv16[r16]
                for d in range(DH // 16):
                    sl = pl.ds(d * 16, 16)
                    fblo[r, sl] = fbuf[r, sl] * sv
                    fbhi[r, sl] = fbuf[r, pl.ds(DH + d * 16, 16)] * sv
            return 0

        lax.fori_loop(0, SCCH // 16, grp, 0)
        pltpu.sync_copy(fblo, f0lo_hbm.at[pl.ds(jbase + c * SCCH, SCCH), :])
        pltpu.sync_copy(fbhi, f0hi_hbm.at[pl.ds(jbase + c * SCCH, SCCH), :])
        return 0

    lax.fori_loop(0, NSC, f0_chunk, 0)


@functools.partial(
    pl.kernel,
    out_type=(
        jax.ShapeDtypeStruct((NC * NPAD, DH), F32),
        jax.ShapeDtypeStruct((NC * NPAD, DH), F32),
    ),
    mesh=MESH,
    compiler_params=CPARAMS,
    scratch_types=[
        pltpu.VMEM_SHARED((NPAD, DH), F32),
        pltpu.VMEM_SHARED((NPAD, DH), F32),
        pltpu.VMEM((SB, CH), I32),
        pltpu.VMEM((SB, CH), I32),
        pltpu.VMEM((CH, DH), F32),
        pltpu.VMEM((CH, DH), F32),
        pltpu.VMEM((CH, DH), F32),
        pltpu.VMEM((CH, DH), F32),
        pltpu.VMEM((RT,), F32),
        pltpu.SemaphoreType.DMA,
        pltpu.SemaphoreType.DMA,
        pltpu.SemaphoreType.DMA,
        pltpu.SemaphoreType.DMA,
    ],
)
def _k_layer(rows_hbm, cols_hbm, flo_in, fhi_in, s_hbm, flo_out, fhi_out,
             src_sp, g_sp, idxr, idxc, m0, m1, m2, m3, sbuf, s0, s1, s2, s3):
    cid = lax.axis_index("c")
    sid = lax.axis_index("s")
    rbase = sid * RT
    cbase = sid * CPT
    ms = (m0, m1, m2, m3)
    sems = (s0, s1, s2, s3)
    jbase = cid * NPAD + rbase
    obase = (1 - cid) * NPAD + rbase

    pltpu.sync_copy(s_hbm.at[pl.ds(jbase, RT)], sbuf)

    for h, (f_in, f_out) in enumerate(((flo_in, flo_out), (fhi_in, fhi_out))):
        pltpu.sync_copy(f_in.at[pl.ds(obase, RT), :],
                        src_sp.at[pl.ds(rbase, RT), :])

        def fz(r, _):
            for d in range(DH // 16):
                m0[r, pl.ds(d * 16, 16)] = jnp.zeros((16,), F32)
            return 0

        lax.fori_loop(0, SCCH, fz, 0)

        def zc(c, _):
            pltpu.sync_copy(m0.at[pl.ds(0, SCCH), :],
                            g_sp.at[pl.ds(rbase + c * SCCH, SCCH), :])
            return 0

        lax.fori_loop(0, NSC, zc, 0)
        plsc.subcore_barrier()

        for u in range(NSUP):
            off = cbase + u * SB
            pltpu.sync_copy(rows_hbm.at[cid, pl.ds(off, SB)], idxr)
            pltpu.sync_copy(cols_hbm.at[cid, pl.ds(off, SB)], idxc)

            def edge_loop(t, _):
                descs = []
                for b in range(NBUF):
                    j = t * NBUF + b
                    descs.append(
                        pltpu.async_copy(src_sp.at[idxc.at[j]], ms[b], sems[b]))
                for b in range(NBUF):
                    j = t * NBUF + b
                    descs[b].wait()
                    pltpu.sync_copy(ms[b], g_sp.at[idxr.at[j]], add=True)
                return 0

            lax.fori_loop(0, SB // NBUF, edge_loop, 0)
        plsc.subcore_barrier()

        def scale_chunk(c, _):
            pltpu.sync_copy(g_sp.at[pl.ds(rbase + c * SCCH, SCCH), :],
                            m0.at[pl.ds(0, SCCH), :])

            def grp(g, _):
                sv16 = sbuf[pl.ds(c * SCCH + g * 16, 16)]
                dv16 = sv16 * sv16
                for r16 in range(16):
                    r = g * 16 + r16
                    dv = dv16[r16]
                    for d in range(DH // 16):
                        sl = pl.ds(d * 16, 16)
                        m0[r, sl] = m0[r, sl] * dv
                return 0

            lax.fori_loop(0, SCCH // 16, grp, 0)
            pltpu.sync_copy(m0.at[pl.ds(0, SCCH), :],
                            f_out.at[pl.ds(jbase + c * SCCH, SCCH), :])
            return 0

        lax.fori_loop(0, NSC, scale_chunk, 0)
        plsc.subcore_barrier()


@functools.partial(
    pl.kernel,
    out_type=jax.ShapeDtypeStruct((B,), F32),
    mesh=MESH,
    compiler_params=CPARAMS,
    scratch_types=[
        pltpu.VMEM((PPT,), I32),
        pltpu.VMEM((PPT,), I32),
        pltpu.VMEM((PPT, DH), F32),
        pltpu.VMEM((PPT, DH), F32),
        pltpu.VMEM((PPT, DH), F32),
        pltpu.VMEM((PPT, DH), F32),
        pltpu.VMEM((PPT,), F32),
        pltpu.VMEM((PPT,), F32),
        pltpu.VMEM((PPT,), F32),
    ],
)
def _k_decode(f0lo, f0hi, f1lo, f1hi, f2lo, f2hi, f3lo, f3hi,
              s_flat, uj_hbm, ij_hbm, gamma,
              uidx, iidx, bul, buh, bil, bih, su, si, gbuf):
    cid = lax.axis_index("c")
    sid = lax.axis_index("s")
    base = (cid * NS + sid) * PPT
    pltpu.sync_copy(uj_hbm.at[pl.ds(base, PPT)], uidx)
    pltpu.sync_copy(ij_hbm.at[pl.ds(base, PPT)], iidx)
    for tab, buf, idx in (
        (f0lo, bul, uidx), (f0hi, buh, uidx),
        (f0lo, bil, iidx), (f0hi, bih, iidx),
    ):
        pltpu.sync_copy(tab.at[idx], buf)
    for tab, buf, idx in (
        (f1lo, bul, uidx), (f2lo, bul, uidx), (f3lo, bul, uidx),
        (f1hi, buh, uidx), (f2hi, buh, uidx), (f3hi, buh, uidx),
        (f1lo, bil, iidx), (f2lo, bil, iidx), (f3lo, bil, iidx),
        (f1hi, bih, iidx), (f2hi, bih, iidx), (f3hi, bih, iidx),
    ):
        pltpu.sync_copy(tab.at[idx], buf, add=True)
    pltpu.sync_copy(s_flat.at[uidx], su)
    pltpu.sync_copy(s_flat.at[iidx], si)

    def grp(g, _):
        rows = lax.iota(I32, 16) + g * 16
        acc = jnp.zeros((16,), F32)
        for d in range(DH):
            cols = jnp.full((16,), d, I32)
            acc = acc + (plsc.load_gather(bul, [rows, cols])
                         * plsc.load_gather(bil, [rows, cols]))
            acc = acc + (plsc.load_gather(buh, [rows, cols])
                         * plsc.load_gather(bih, [rows, cols]))
        sl = pl.ds(g * 16, 16)
        gbuf[sl] = acc / (su[sl] * si[sl] * 16.0)
        return 0

    lax.fori_loop(0, PPT // 16, grp, 0)
    pltpu.sync_copy(gbuf, gamma.at[pl.ds(base, PPT)])


@jax.jit
def kernel(user_emb, item_emb, edge_index, users, items):
    src = edge_index[0].astype(I32)
    dst = edge_index[1].astype(I32)
    padr = jnp.full((EPAD - E,), JUNK, I32)
    padc = jnp.zeros((EPAD - E,), I32)
    rows3d = jnp.stack([
        jnp.concatenate([src, padr]),
        jnp.concatenate([dst, padr]),
    ]).reshape(NC, NS * CPT, CH)
    cols3d = jnp.stack([
        jnp.concatenate([dst, padc]),
        jnp.concatenate([src, padc]),
    ]).reshape(NC, NS * CPT, CH)
    zpad = jnp.zeros((NPAD - NU, D), F32)
    e0p = jnp.concatenate([user_emb, zpad, item_emb, zpad], axis=0)

    s1d, f0lo, f0hi = _k_deg(rows3d, e0p)
    f1lo, f1hi = _k_layer(rows3d, cols3d, f0lo, f0hi, s1d)
    f2lo, f2hi = _k_layer(rows3d, cols3d, f1lo, f1hi, s1d)
    f3lo, f3hi = _k_layer(rows3d, cols3d, f2lo, f2hi, s1d)
    gamma = _k_decode(f0lo, f0hi, f1lo, f1hi, f2lo, f2hi, f3lo, f3hi,
                      s1d, users.astype(I32), (items.astype(I32) + NPAD))
    return gamma


# speedup vs baseline: 1.2989x; 1.0216x over previous
import functools

import jax
import jax.numpy as jnp
from jax import lax
from jax.experimental import pallas as pl
from jax.experimental.pallas import tpu as pltpu
from jax.experimental.pallas import tpu_sc as plsc

NU = 25000
D = 64
DH = D // 2
E = 400000
B = 4096
NC = 2
NS = 16
RT = 1568
NPAD = NS * RT
JUNK = NU
CH = 128
CPT = 200
EPAD = CPT * CH * NS
SB = 20
NSUP = CPT // SB
NBUF = 4
SCCH = 112
NSC = RT // SCCH
PPT = B // (NC * NS)

F32 = jnp.float32
I32 = jnp.int32

MESH = plsc.VectorSubcoreMesh(
    core_axis_name="c", subcore_axis_name="s", num_cores=NC, num_subcores=NS
)
CPARAMS = pltpu.CompilerParams(use_tc_tiling_on_sc=False, needs_layout_passes=False)


def _rsqrt_newton(x):
    i = lax.bitcast_convert_type(x, I32)
    i = 0x5F3759DF - jnp.right_shift(i, 1)
    y = lax.bitcast_convert_type(i, F32)
    for _ in range(3):
        y = y * (1.5 - 0.5 * x * y * y)
    return y


@functools.partial(
    pl.kernel,
    out_type=(
        jax.ShapeDtypeStruct((NC * NPAD,), F32),
        jax.ShapeDtypeStruct((NC * NPAD, DH), F32),
        jax.ShapeDtypeStruct((NC * NPAD, DH), F32),
    ),
    mesh=MESH,
    compiler_params=CPARAMS,
    scratch_types=[
        pltpu.VMEM_SHARED((NPAD,), F32),
        pltpu.VMEM((CPT, CH), I32),
        pltpu.VMEM((RT,), F32),
        pltpu.VMEM((RT,), F32),
        pltpu.VMEM((CH,), F32),
        pltpu.VMEM((SCCH, D), F32),
        pltpu.VMEM((SCCH, DH), F32),
        pltpu.VMEM((SCCH, DH), F32),
        pltpu.SemaphoreType.DMA,
    ],
)
def _k_deg(rows_hbm, e0_hbm, s_hbm, f0lo_hbm, f0hi_hbm,
           deg_sp, idxr, zbuf, sbuf, ones, fbuf, fblo, fbhi, sdeg):
    cid = lax.axis_index("c")
    sid = lax.axis_index("s")
    rbase = sid * RT

    def fz(i, _):
        zbuf[pl.ds(i * 16, 16)] = jnp.zeros((16,), F32)
        return 0

    lax.fori_loop(0, RT // 16, fz, 0)
    for i in range(CH // 16):
        ones[pl.ds(i * 16, 16)] = jnp.ones((16,), F32)
    pltpu.sync_copy(zbuf, deg_sp.at[pl.ds(rbase, RT)])
    pltpu.sync_copy(rows_hbm.at[cid, pl.ds(sid * CPT, CPT)], idxr)
    plsc.subcore_barrier()

    def deg_add(t, _):
        descs = [
            pltpu.async_copy(ones, deg_sp.at[idxr.at[t * 8 + b]], sdeg, add=True)
            for b in range(8)
        ]
        for dsc in descs:
            dsc.wait()
        return 0

    lax.fori_loop(0, CPT // 8, deg_add, 0)
    plsc.subcore_barrier()

    pltpu.sync_copy(deg_sp.at[pl.ds(rbase, RT)], zbuf)

    def newton(i, _):
        x = jnp.maximum(zbuf[pl.ds(i * 16, 16)], 1.0)
        sbuf[pl.ds(i * 16, 16)] = _rsqrt_newton(x)
        return 0

    lax.fori_loop(0, RT // 16, newton, 0)
    pltpu.sync_copy(sbuf, s_hbm.at[pl.ds(cid * NPAD + rbase, RT)])

    jbase = cid * NPAD + rbase

    def f0_chunk(c, _):
        pltpu.sync_copy(e0_hbm.at[pl.ds(jbase + c * SCCH, SCCH), :], fbuf)

        def grp(g, _):
            sv16 = sbuf[pl.ds(c * SCCH + g * 16, 16)]
            for r16 in range(16):
                r = g * 16 + r16
                sv = sv16[r16]
                for d in range(DH // 16):
                    sl = pl.ds(d * 16, 16)
                    fblo[r, sl] = fbuf[r, sl] * sv
                    fbhi[r, sl] = fbuf[r, pl.ds(DH + d * 16, 16)] * sv
            return 0

        lax.fori_loop(0, SCCH // 16, grp, 0)
        pltpu.sync_copy(fblo, f0lo_hbm.at[pl.ds(jbase + c * SCCH, SCCH), :])
        pltpu.sync_copy(fbhi, f0hi_hbm.at[pl.ds(jbase + c * SCCH, SCCH), :])
        return 0

    lax.fori_loop(0, NSC, f0_chunk, 0)


@functools.partial(
    pl.kernel,
    out_type=(
        jax.ShapeDtypeStruct((NC * NPAD, DH), F32),
        jax.ShapeDtypeStruct((NC * NPAD, DH), F32),
    ),
    mesh=MESH,
    compiler_params=CPARAMS,
    scratch_types=[
        pltpu.VMEM_SHARED((NPAD, DH), F32),
        pltpu.VMEM_SHARED((NPAD, DH), F32),
        pltpu.VMEM((2, SB, CH), I32),
        pltpu.VMEM((2, SB, CH), I32),
        pltpu.VMEM((CH, DH), F32),
        pltpu.VMEM((CH, DH), F32),
        pltpu.VMEM((CH, DH), F32),
        pltpu.VMEM((CH, DH), F32),
        pltpu.VMEM((RT,), F32),
        pltpu.SemaphoreType.DMA,
        pltpu.SemaphoreType.DMA,
        pltpu.SemaphoreType.DMA,
        pltpu.SemaphoreType.DMA,
        pltpu.SemaphoreType.DMA,
        pltpu.SemaphoreType.DMA,
    ],
)
def _k_layer(rows_hbm, cols_hbm, flo_in, fhi_in, s_hbm, flo_out, fhi_out,
             src_sp, g_sp, idxr, idxc, m0, m1, m2, m3, sbuf,
             s0, s1, s2, s3, sir, sic):
    cid = lax.axis_index("c")
    sid = lax.axis_index("s")
    rbase = sid * RT
    cbase = sid * CPT
    ms = (m0, m1, m2, m3)
    sems = (s0, s1, s2, s3)
    jbase = cid * NPAD + rbase
    obase = (1 - cid) * NPAD + rbase

    pltpu.sync_copy(s_hbm.at[pl.ds(jbase, RT)], sbuf)

    for h, (f_in, f_out) in enumerate(((flo_in, flo_out), (fhi_in, fhi_out))):
        pltpu.sync_copy(f_in.at[pl.ds(obase, RT), :],
                        src_sp.at[pl.ds(rbase, RT), :])

        def fz(r, _):
            for d in range(DH // 16):
                m0[r, pl.ds(d * 16, 16)] = jnp.zeros((16,), F32)
            return 0

        lax.fori_loop(0, SCCH, fz, 0)

        def zc(c, _):
            pltpu.sync_copy(m0.at[pl.ds(0, SCCH), :],
                            g_sp.at[pl.ds(rbase + c * SCCH, SCCH), :])
            return 0

        lax.fori_loop(0, NSC, zc, 0)
        plsc.subcore_barrier()

        pltpu.sync_copy(rows_hbm.at[cid, pl.ds(cbase, SB)], idxr.at[0])
        pltpu.sync_copy(cols_hbm.at[cid, pl.ds(cbase, SB)], idxc.at[0])
        for u in range(NSUP):
            cur = u % 2
            if u + 1 < NSUP:
                off = cbase + (u + 1) * SB
                pr = pltpu.async_copy(rows_hbm.at[cid, pl.ds(off, SB)],
                                      idxr.at[1 - cur], sir)
                pc = pltpu.async_copy(cols_hbm.at[cid, pl.ds(off, SB)],
                                      idxc.at[1 - cur], sic)

            def edge_loop(t, _):
                descs = []
                for b in range(NBUF):
                    j = t * NBUF + b
                    descs.append(
                        pltpu.async_copy(src_sp.at[idxc.at[cur, j]],
                                         ms[b], sems[b]))
                for b in range(NBUF):
                    j = t * NBUF + b
                    descs[b].wait()
                    pltpu.sync_copy(ms[b], g_sp.at[idxr.at[cur, j]], add=True)
                return 0

            lax.fori_loop(0, SB // NBUF, edge_loop, 0)
            if u + 1 < NSUP:
                pr.wait()
                pc.wait()
        plsc.subcore_barrier()

        def scale_chunk(c, _):
            pltpu.sync_copy(g_sp.at[pl.ds(rbase + c * SCCH, SCCH), :],
                            m0.at[pl.ds(0, SCCH), :])

            def grp(g, _):
                sv16 = sbuf[pl.ds(c * SCCH + g * 16, 16)]
                dv16 = sv16 * sv16
                for r16 in range(16):
                    r = g * 16 + r16
                    dv = dv16[r16]
                    for d in range(DH // 16):
                        sl = pl.ds(d * 16, 16)
                        m0[r, sl] = m0[r, sl] * dv
                return 0

            lax.fori_loop(0, SCCH // 16, grp, 0)
            pltpu.sync_copy(m0.at[pl.ds(0, SCCH), :],
                            f_out.at[pl.ds(jbase + c * SCCH, SCCH), :])
            return 0

        lax.fori_loop(0, NSC, scale_chunk, 0)
        plsc.subcore_barrier()


@functools.partial(
    pl.kernel,
    out_type=jax.ShapeDtypeStruct((B,), F32),
    mesh=MESH,
    compiler_params=CPARAMS,
    scratch_types=[
        pltpu.VMEM((PPT,), I32),
        pltpu.VMEM((PPT,), I32),
        pltpu.VMEM((PPT, DH), F32),
        pltpu.VMEM((PPT, DH), F32),
        pltpu.VMEM((PPT, DH), F32),
        pltpu.VMEM((PPT, DH), F32),
        pltpu.VMEM((PPT,), F32),
        pltpu.VMEM((PPT,), F32),
        pltpu.VMEM((PPT,), F32),
    ],
)
def _k_decode(f0lo, f0hi, f1lo, f1hi, f2lo, f2hi, f3lo, f3hi,
              s_flat, uj_hbm, ij_hbm, gamma,
              uidx, iidx, bul, buh, bil, bih, su, si, gbuf):
    cid = lax.axis_index("c")
    sid = lax.axis_index("s")
    base = (cid * NS + sid) * PPT
    pltpu.sync_copy(uj_hbm.at[pl.ds(base, PPT)], uidx)
    pltpu.sync_copy(ij_hbm.at[pl.ds(base, PPT)], iidx)
    for tab, buf, idx in (
        (f0lo, bul, uidx), (f0hi, buh, uidx),
        (f0lo, bil, iidx), (f0hi, bih, iidx),
    ):
        pltpu.sync_copy(tab.at[idx], buf)
    for tab, buf, idx in (
        (f1lo, bul, uidx), (f2lo, bul, uidx), (f3lo, bul, uidx),
        (f1hi, buh, uidx), (f2hi, buh, uidx), (f3hi, buh, uidx),
        (f1lo, bil, iidx), (f2lo, bil, iidx), (f3lo, bil, iidx),
        (f1hi, bih, iidx), (f2hi, bih, iidx), (f3hi, bih, iidx),
    ):
        pltpu.sync_copy(tab.at[idx], buf, add=True)
    pltpu.sync_copy(s_flat.at[uidx], su)
    pltpu.sync_copy(s_flat.at[iidx], si)

    def grp(g, _):
        rows = lax.iota(I32, 16) + g * 16
        acc = jnp.zeros((16,), F32)
        for d in range(DH):
            cols = jnp.full((16,), d, I32)
            acc = acc + (plsc.load_gather(bul, [rows, cols])
                         * plsc.load_gather(bil, [rows, cols]))
            acc = acc + (plsc.load_gather(buh, [rows, cols])
                         * plsc.load_gather(bih, [rows, cols]))
        sl = pl.ds(g * 16, 16)
        gbuf[sl] = acc / (su[sl] * si[sl] * 16.0)
        return 0

    lax.fori_loop(0, PPT // 16, grp, 0)
    pltpu.sync_copy(gbuf, gamma.at[pl.ds(base, PPT)])


@jax.jit
def kernel(user_emb, item_emb, edge_index, users, items):
    src = edge_index[0].astype(I32)
    dst = edge_index[1].astype(I32)
    padr = jnp.full((EPAD - E,), JUNK, I32)
    padc = jnp.zeros((EPAD - E,), I32)
    rows3d = jnp.stack([
        jnp.concatenate([src, padr]),
        jnp.concatenate([dst, padr]),
    ]).reshape(NC, NS * CPT, CH)
    cols3d = jnp.stack([
        jnp.concatenate([dst, padc]),
        jnp.concatenate([src, padc]),
    ]).reshape(NC, NS * CPT, CH)
    zpad = jnp.zeros((NPAD - NU, D), F32)
    e0p = jnp.concatenate([user_emb, zpad, item_emb, zpad], axis=0)

    s1d, f0lo, f0hi = _k_deg(rows3d, e0p)
    f1lo, f1hi = _k_layer(rows3d, cols3d, f0lo, f0hi, s1d)
    f2lo, f2hi = _k_layer(rows3d, cols3d, f1lo, f1hi, s1d)
    f3lo, f3hi = _k_layer(rows3d, cols3d, f2lo, f2hi, s1d)
    gamma = _k_decode(f0lo, f0hi, f1lo, f1hi, f2lo, f2hi, f3lo, f3hi,
                      s1d, users.astype(I32), (items.astype(I32) + NPAD))
    return gamma


# speedup vs baseline: 1.3031x; 1.0032x over previous
import functools

import jax
import jax.numpy as jnp
from jax import lax
from jax.experimental import pallas as pl
from jax.experimental.pallas import tpu as pltpu
from jax.experimental.pallas import tpu_sc as plsc

NU = 25000
D = 64
DH = D // 2
E = 400000
B = 4096
NC = 2
NS = 16
RT = 1568
NPAD = NS * RT
JUNK = NU
CH = 128
CPT = 200
EPAD = CPT * CH * NS
SB = 20
NSUP = CPT // SB
NBUF = 4
SCCH = 112
NSC = RT // SCCH
PPT = B // (NC * NS)

F32 = jnp.float32
I32 = jnp.int32

MESH = plsc.VectorSubcoreMesh(
    core_axis_name="c", subcore_axis_name="s", num_cores=NC, num_subcores=NS
)
CPARAMS = pltpu.CompilerParams(use_tc_tiling_on_sc=False, needs_layout_passes=False)


def _rsqrt_newton(x):
    i = lax.bitcast_convert_type(x, I32)
    i = 0x5F3759DF - jnp.right_shift(i, 1)
    y = lax.bitcast_convert_type(i, F32)
    for _ in range(3):
        y = y * (1.5 - 0.5 * x * y * y)
    return y


@functools.partial(
    pl.kernel,
    out_type=(
        jax.ShapeDtypeStruct((NC * NPAD,), F32),
        jax.ShapeDtypeStruct((NC * NPAD, DH), F32),
        jax.ShapeDtypeStruct((NC * NPAD, DH), F32),
    ),
    mesh=MESH,
    compiler_params=CPARAMS,
    scratch_types=[
        pltpu.VMEM_SHARED((NPAD,), F32),
        pltpu.VMEM((CPT, CH), I32),
        pltpu.VMEM((RT,), F32),
        pltpu.VMEM((RT,), F32),
        pltpu.VMEM((CH,), F32),
        pltpu.VMEM((SCCH, D), F32),
        pltpu.VMEM((SCCH, DH), F32),
        pltpu.VMEM((SCCH, DH), F32),
        pltpu.SemaphoreType.DMA,
    ],
)
def _k_deg(rows_hbm, e0_hbm, s_hbm, f0lo_hbm, f0hi_hbm,
           deg_sp, idxr, zbuf, sbuf, ones, fbuf, fblo, fbhi, sdeg):
    cid = lax.axis_index("c")
    sid = lax.axis_index("s")
    rbase = sid * RT

    def fz(i, _):
        zbuf[pl.ds(i * 16, 16)] = jnp.zeros((16,), F32)
        return 0

    lax.fori_loop(0, RT // 16, fz, 0)
    for i in range(CH // 16):
        ones[pl.ds(i * 16, 16)] = jnp.ones((16,), F32)
    pltpu.sync_copy(zbuf, deg_sp.at[pl.ds(rbase, RT)])
    pltpu.sync_copy(rows_hbm.at[cid, pl.ds(sid * CPT, CPT)], idxr)
    plsc.subcore_barrier()

    def deg_add(t, _):
        descs = [
            pltpu.async_copy(ones, deg_sp.at[idxr.at[t * 8 + b]], sdeg, add=True)
            for b in range(8)
        ]
        for dsc in descs:
            dsc.wait()
        return 0

    lax.fori_loop(0, CPT // 8, deg_add, 0)
    plsc.subcore_barrier()

    pltpu.sync_copy(deg_sp.at[pl.ds(rbase, RT)], zbuf)

    def newton(i, _):
        x = jnp.maximum(zbuf[pl.ds(i * 16, 16)], 1.0)
        sbuf[pl.ds(i * 16, 16)] = _rsqrt_newton(x)
        return 0

    lax.fori_loop(0, RT // 16, newton, 0)
    pltpu.sync_copy(sbuf, s_hbm.at[pl.ds(cid * NPAD + rbase, RT)])

    jbase = cid * NPAD + rbase

    def f0_chunk(c, _):
        pltpu.sync_copy(e0_hbm.at[pl.ds(jbase + c * SCCH, SCCH), :], fbuf)

        def grp(g, _):
            sv16 = sbuf[pl.ds(c * SCCH + g * 16, 16)]
            for r16 in range(16):
                r = g * 16 + r16
                sv = sv16[r16]
                for d in range(DH // 16):
                    sl = pl.ds(d * 16, 16)
                    fblo[r, sl] = fbuf[r, sl] * sv
                    fbhi[r, sl] = fbuf[r, pl.ds(DH + d * 16, 16)] * sv
            return 0

        lax.fori_loop(0, SCCH // 16, grp, 0)
        pltpu.sync_copy(fblo, f0lo_hbm.at[pl.ds(jbase + c * SCCH, SCCH), :])
        pltpu.sync_copy(fbhi, f0hi_hbm.at[pl.ds(jbase + c * SCCH, SCCH), :])
        return 0

    lax.fori_loop(0, NSC, f0_chunk, 0)


@functools.partial(
    pl.kernel,
    out_type=(
        jax.ShapeDtypeStruct((NC * NPAD, DH), F32),
        jax.ShapeDtypeStruct((NC * NPAD, DH), F32),
    ),
    mesh=MESH,
    compiler_params=CPARAMS,
    scratch_types=[
        pltpu.VMEM_SHARED((NPAD, DH), F32),
        pltpu.VMEM_SHARED((NPAD, DH), F32),
        pltpu.VMEM((2, SB, CH), I32),
        pltpu.VMEM((2, SB, CH), I32),
        pltpu.VMEM((CH, DH), F32),
        pltpu.VMEM((CH, DH), F32),
        pltpu.VMEM((CH, DH), F32),
        pltpu.VMEM((CH, DH), F32),
        pltpu.VMEM((RT,), F32),
        pltpu.SemaphoreType.DMA,
        pltpu.SemaphoreType.DMA,
        pltpu.SemaphoreType.DMA,
        pltpu.SemaphoreType.DMA,
        pltpu.SemaphoreType.DMA,
        pltpu.SemaphoreType.DMA,
        pltpu.SemaphoreType.DMA,
        pltpu.SemaphoreType.DMA,
        pltpu.SemaphoreType.DMA,
        pltpu.SemaphoreType.DMA,
    ],
)
def _k_layer(rows_hbm, cols_hbm, flo_in, fhi_in, s_hbm, flo_out, fhi_out,
             src_sp, g_sp, idxr, idxc, m0, m1, m2, m3, sbuf,
             s0, s1, s2, s3, t0, t1, t2, t3, sir, sic):
    cid = lax.axis_index("c")
    sid = lax.axis_index("s")
    rbase = sid * RT
    cbase = sid * CPT
    ms = (m0, m1, m2, m3)
    gsems = (s0, s1, s2, s3)
    ssems = (t0, t1, t2, t3)
    jbase = cid * NPAD + rbase
    obase = (1 - cid) * NPAD + rbase

    pltpu.sync_copy(s_hbm.at[pl.ds(jbase, RT)], sbuf)

    for h, (f_in, f_out) in enumerate(((flo_in, flo_out), (fhi_in, fhi_out))):
        pltpu.sync_copy(f_in.at[pl.ds(obase, RT), :],
                        src_sp.at[pl.ds(rbase, RT), :])

        def fz(r, _):
            for d in range(DH // 16):
                m0[r, pl.ds(d * 16, 16)] = jnp.zeros((16,), F32)
            return 0

        lax.fori_loop(0, SCCH, fz, 0)

        def zc(c, _):
            pltpu.sync_copy(m0.at[pl.ds(0, SCCH), :],
                            g_sp.at[pl.ds(rbase + c * SCCH, SCCH), :])
            return 0

        lax.fori_loop(0, NSC, zc, 0)
        plsc.subcore_barrier()

        pltpu.sync_copy(rows_hbm.at[cid, pl.ds(cbase, SB)], idxr.at[0])
        pltpu.sync_copy(cols_hbm.at[cid, pl.ds(cbase, SB)], idxc.at[0])
        for u in range(NSUP):
            cur = u % 2
            if u + 1 < NSUP:
                off = cbase + (u + 1) * SB
                pr = pltpu.async_copy(rows_hbm.at[cid, pl.ds(off, SB)],
                                      idxr.at[1 - cur], sir)
                pc = pltpu.async_copy(cols_hbm.at[cid, pl.ds(off, SB)],
                                      idxc.at[1 - cur], sic)

            gd = [None] * NBUF
            sd = [None] * NBUF
            for jl in range(SB):
                b = jl % NBUF
                if sd[b] is not None:
                    sd[b].wait()
                    sd[b] = None
                gd[b] = pltpu.async_copy(src_sp.at[idxc.at[cur, jl]],
                                         ms[b], gsems[b])
                jp = jl - (NBUF - 1)
                if jp >= 0:
                    bp = jp % NBUF
                    gd[bp].wait()
                    sd[bp] = pltpu.async_copy(
                        ms[bp], g_sp.at[idxr.at[cur, jp]], ssems[bp], add=True)
            for jp in range(SB - NBUF + 1, SB):
                bp = jp % NBUF
                gd[bp].wait()
                sd[bp] = pltpu.async_copy(
                    ms[bp], g_sp.at[idxr.at[cur, jp]], ssems[bp], add=True)
            for b in range(NBUF):
                if sd[b] is not None:
                    sd[b].wait()
            if u + 1 < NSUP:
                pr.wait()
                pc.wait()
        plsc.subcore_barrier()

        def scale_chunk(c, _):
            pltpu.sync_copy(g_sp.at[pl.ds(rbase + c * SCCH, SCCH), :],
                            m0.at[pl.ds(0, SCCH), :])

            def grp(g, _):
                sv16 = sbuf[pl.ds(c * SCCH + g * 16, 16)]
                dv16 = sv16 * sv16
                for r16 in range(16):
                    r = g * 16 + r16
                    dv = dv16[r16]
                    for d in range(DH // 16):
                        sl = pl.ds(d * 16, 16)
                        m0[r, sl] = m0[r, sl] * dv
                return 0

            lax.fori_loop(0, SCCH // 16, grp, 0)
            pltpu.sync_copy(m0.at[pl.ds(0, SCCH), :],
                            f_out.at[pl.ds(jbase + c * SCCH, SCCH), :])
            return 0

        lax.fori_loop(0, NSC, scale_chunk, 0)
        plsc.subcore_barrier()


@functools.partial(
    pl.kernel,
    out_type=jax.ShapeDtypeStruct((B,), F32),
    mesh=MESH,
    compiler_params=CPARAMS,
    scratch_types=[
        pltpu.VMEM((PPT,), I32),
        pltpu.VMEM((PPT,), I32),
        pltpu.VMEM((PPT, DH), F32),
        pltpu.VMEM((PPT, DH), F32),
        pltpu.VMEM((PPT, DH), F32),
        pltpu.VMEM((PPT, DH), F32),
        pltpu.VMEM((PPT,), F32),
        pltpu.VMEM((PPT,), F32),
        pltpu.VMEM((PPT,), F32),
    ],
)
def _k_decode(f0lo, f0hi, f1lo, f1hi, f2lo, f2hi, f3lo, f3hi,
              s_flat, uj_hbm, ij_hbm, gamma,
              uidx, iidx, bul, buh, bil, bih, su, si, gbuf):
    cid = lax.axis_index("c")
    sid = lax.axis_index("s")
    base = (cid * NS + sid) * PPT
    pltpu.sync_copy(uj_hbm.at[pl.ds(base, PPT)], uidx)
    pltpu.sync_copy(ij_hbm.at[pl.ds(base, PPT)], iidx)
    for tab, buf, idx in (
        (f0lo, bul, uidx), (f0hi, buh, uidx),
        (f0lo, bil, iidx), (f0hi, bih, iidx),
    ):
        pltpu.sync_copy(tab.at[idx], buf)
    for tab, buf, idx in (
        (f1lo, bul, uidx), (f2lo, bul, uidx), (f3lo, bul, uidx),
        (f1hi, buh, uidx), (f2hi, buh, uidx), (f3hi, buh, uidx),
        (f1lo, bil, iidx), (f2lo, bil, iidx), (f3lo, bil, iidx),
        (f1hi, bih, iidx), (f2hi, bih, iidx), (f3hi, bih, iidx),
    ):
        pltpu.sync_copy(tab.at[idx], buf, add=True)
    pltpu.sync_copy(s_flat.at[uidx], su)
    pltpu.sync_copy(s_flat.at[iidx], si)

    def grp(g, _):
        rows = lax.iota(I32, 16) + g * 16
        acc = jnp.zeros((16,), F32)
        for d in range(DH):
            cols = jnp.full((16,), d, I32)
            acc = acc + (plsc.load_gather(bul, [rows, cols])
                         * plsc.load_gather(bil, [rows, cols]))
            acc = acc + (plsc.load_gather(buh, [rows, cols])
                         * plsc.load_gather(bih, [rows, cols]))
        sl = pl.ds(g * 16, 16)
        gbuf[sl] = acc / (su[sl] * si[sl] * 16.0)
        return 0

    lax.fori_loop(0, PPT // 16, grp, 0)
    pltpu.sync_copy(gbuf, gamma.at[pl.ds(base, PPT)])


@jax.jit
def kernel(user_emb, item_emb, edge_index, users, items):
    src = edge_index[0].astype(I32)
    dst = edge_index[1].astype(I32)
    padr = jnp.full((EPAD - E,), JUNK, I32)
    padc = jnp.zeros((EPAD - E,), I32)
    rows3d = jnp.stack([
        jnp.concatenate([src, padr]),
        jnp.concatenate([dst, padr]),
    ]).reshape(NC, NS * CPT, CH)
    cols3d = jnp.stack([
        jnp.concatenate([dst, padc]),
        jnp.concatenate([src, padc]),
    ]).reshape(NC, NS * CPT, CH)
    zpad = jnp.zeros((NPAD - NU, D), F32)
    e0p = jnp.concatenate([user_emb, zpad, item_emb, zpad], axis=0)

    s1d, f0lo, f0hi = _k_deg(rows3d, e0p)
    f1lo, f1hi = _k_layer(rows3d, cols3d, f0lo, f0hi, s1d)
    f2lo, f2hi = _k_layer(rows3d, cols3d, f1lo, f1hi, s1d)
    f3lo, f3hi = _k_layer(rows3d, cols3d, f2lo, f2hi, s1d)
    gamma = _k_decode(f0lo, f0hi, f1lo, f1hi, f2lo, f2hi, f3lo, f3hi,
                      s1d, users.astype(I32), (items.astype(I32) + NPAD))
    return gamma


# speedup vs baseline: 1.3265x; 1.0180x over previous
import functools

import jax
import jax.numpy as jnp
from jax import lax
from jax.experimental import pallas as pl
from jax.experimental.pallas import tpu as pltpu
from jax.experimental.pallas import tpu_sc as plsc

NU = 25000
D = 64
DH = D // 2
E = 400000
B = 4096
NC = 2
NS = 16
RT = 1568
NPAD = NS * RT
JUNK = NU
CH = 128
CPT = 200
EPAD = CPT * CH * NS
SB = 20
NSUP = CPT // SB
NBUF = 4
SCCH = 112
NSC = RT // SCCH
PPT = B // (NC * NS)

F32 = jnp.float32
I32 = jnp.int32

MESH = plsc.VectorSubcoreMesh(
    core_axis_name="c", subcore_axis_name="s", num_cores=NC, num_subcores=NS
)
CPARAMS = pltpu.CompilerParams(use_tc_tiling_on_sc=False, needs_layout_passes=False)


def _rsqrt_newton(x):
    i = lax.bitcast_convert_type(x, I32)
    i = 0x5F3759DF - jnp.right_shift(i, 1)
    y = lax.bitcast_convert_type(i, F32)
    for _ in range(3):
        y = y * (1.5 - 0.5 * x * y * y)
    return y


@functools.partial(
    pl.kernel,
    out_type=(
        jax.ShapeDtypeStruct((NC * NPAD,), F32),
        jax.ShapeDtypeStruct((NC * NPAD, DH), F32),
        jax.ShapeDtypeStruct((NC * NPAD, DH), F32),
    ),
    mesh=MESH,
    compiler_params=CPARAMS,
    scratch_types=[
        pltpu.VMEM_SHARED((NPAD,), F32),
        pltpu.VMEM((CPT, CH), I32),
        pltpu.VMEM((RT,), F32),
        pltpu.VMEM((RT,), F32),
        pltpu.VMEM((CH,), F32),
        pltpu.VMEM((SCCH, D), F32),
        pltpu.VMEM((SCCH, D), F32),
        pltpu.VMEM((SCCH, DH), F32),
        pltpu.VMEM((SCCH, DH), F32),
        pltpu.SemaphoreType.DMA,
        pltpu.SemaphoreType.DMA,
    ],
)
def _k_deg(rows_hbm, e0_hbm, s_hbm, f0lo_hbm, f0hi_hbm,
           deg_sp, idxr, zbuf, sbuf, ones, fbufA, fbufB, fblo, fbhi, sdeg, sfe):
    cid = lax.axis_index("c")
    sid = lax.axis_index("s")
    rbase = sid * RT

    def fz(i, _):
        zbuf[pl.ds(i * 16, 16)] = jnp.zeros((16,), F32)
        return 0

    lax.fori_loop(0, RT // 16, fz, 0)
    for i in range(CH // 16):
        ones[pl.ds(i * 16, 16)] = jnp.ones((16,), F32)
    pltpu.sync_copy(zbuf, deg_sp.at[pl.ds(rbase, RT)])
    pltpu.sync_copy(rows_hbm.at[cid, pl.ds(sid * CPT, CPT)], idxr)
    plsc.subcore_barrier()

    def deg_add(t, _):
        descs = [
            pltpu.async_copy(ones, deg_sp.at[idxr.at[t * 8 + b]], sdeg, add=True)
            for b in range(8)
        ]
        for dsc in descs:
            dsc.wait()
        return 0

    lax.fori_loop(0, CPT // 8, deg_add, 0)
    plsc.subcore_barrier()

    pltpu.sync_copy(deg_sp.at[pl.ds(rbase, RT)], zbuf)

    def newton(i, _):
        x = jnp.maximum(zbuf[pl.ds(i * 16, 16)], 1.0)
        sbuf[pl.ds(i * 16, 16)] = _rsqrt_newton(x)
        return 0

    lax.fori_loop(0, RT // 16, newton, 0)
    pltpu.sync_copy(sbuf, s_hbm.at[pl.ds(cid * NPAD + rbase, RT)])

    jbase = cid * NPAD + rbase

    fbs = (fbufA, fbufB)
    pltpu.sync_copy(e0_hbm.at[pl.ds(jbase, SCCH), :], fbufA)
    for c in range(NSC):
        fb = fbs[c % 2]
        if c + 1 < NSC:
            pf = pltpu.async_copy(
                e0_hbm.at[pl.ds(jbase + (c + 1) * SCCH, SCCH), :],
                fbs[1 - c % 2], sfe)

        def grp(g, _, fb=fb, c=c):
            sv16 = sbuf[pl.ds(c * SCCH + g * 16, 16)]
            for r16 in range(16):
                r = g * 16 + r16
                sv = sv16[r16]
                for d in range(DH // 16):
                    sl = pl.ds(d * 16, 16)
                    fblo[r, sl] = fb[r, sl] * sv
                    fbhi[r, sl] = fb[r, pl.ds(DH + d * 16, 16)] * sv
            return 0

        lax.fori_loop(0, SCCH // 16, grp, 0)
        pltpu.sync_copy(fblo, f0lo_hbm.at[pl.ds(jbase + c * SCCH, SCCH), :])
        pltpu.sync_copy(fbhi, f0hi_hbm.at[pl.ds(jbase + c * SCCH, SCCH), :])
        if c + 1 < NSC:
            pf.wait()


@functools.partial(
    pl.kernel,
    out_type=(
        jax.ShapeDtypeStruct((NC * NPAD, DH), F32),
        jax.ShapeDtypeStruct((NC * NPAD, DH), F32),
    ),
    mesh=MESH,
    compiler_params=CPARAMS,
    scratch_types=[
        pltpu.VMEM_SHARED((NPAD, DH), F32),
        pltpu.VMEM_SHARED((NPAD, DH), F32),
        pltpu.VMEM((2, SB, CH), I32),
        pltpu.VMEM((2, SB, CH), I32),
        pltpu.VMEM((CH, DH), F32),
        pltpu.VMEM((CH, DH), F32),
        pltpu.VMEM((CH, DH), F32),
        pltpu.VMEM((CH, DH), F32),
        pltpu.VMEM((RT,), F32),
        pltpu.SemaphoreType.DMA,
        pltpu.SemaphoreType.DMA,
        pltpu.SemaphoreType.DMA,
        pltpu.SemaphoreType.DMA,
        pltpu.SemaphoreType.DMA,
        pltpu.SemaphoreType.DMA,
        pltpu.SemaphoreType.DMA,
        pltpu.SemaphoreType.DMA,
        pltpu.SemaphoreType.DMA,
        pltpu.SemaphoreType.DMA,
    ],
)
def _k_layer(rows_hbm, cols_hbm, flo_in, fhi_in, s_hbm, flo_out, fhi_out,
             src_sp, g_sp, idxr, idxc, m0, m1, m2, m3, sbuf,
             s0, s1, s2, s3, t0, t1, t2, t3, sir, sic):
    cid = lax.axis_index("c")
    sid = lax.axis_index("s")
    rbase = sid * RT
    cbase = sid * CPT
    ms = (m0, m1, m2, m3)
    gsems = (s0, s1, s2, s3)
    ssems = (t0, t1, t2, t3)
    jbase = cid * NPAD + rbase
    obase = (1 - cid) * NPAD + rbase

    pltpu.sync_copy(s_hbm.at[pl.ds(jbase, RT)], sbuf)

    for h, (f_in, f_out) in enumerate(((flo_in, flo_out), (fhi_in, fhi_out))):
        pltpu.sync_copy(f_in.at[pl.ds(obase, RT), :],
                        src_sp.at[pl.ds(rbase, RT), :])

        def fz(r, _):
            for d in range(DH // 16):
                m0[r, pl.ds(d * 16, 16)] = jnp.zeros((16,), F32)
            return 0

        lax.fori_loop(0, SCCH, fz, 0)

        def zc(c, _):
            pltpu.sync_copy(m0.at[pl.ds(0, SCCH), :],
                            g_sp.at[pl.ds(rbase + c * SCCH, SCCH), :])
            return 0

        lax.fori_loop(0, NSC, zc, 0)
        plsc.subcore_barrier()

        pltpu.sync_copy(rows_hbm.at[cid, pl.ds(cbase, SB)], idxr.at[0])
        pltpu.sync_copy(cols_hbm.at[cid, pl.ds(cbase, SB)], idxc.at[0])
        for u in range(NSUP):
            cur = u % 2
            if u + 1 < NSUP:
                off = cbase + (u + 1) * SB
                pr = pltpu.async_copy(rows_hbm.at[cid, pl.ds(off, SB)],
                                      idxr.at[1 - cur], sir)
                pc = pltpu.async_copy(cols_hbm.at[cid, pl.ds(off, SB)],
                                      idxc.at[1 - cur], sic)

            gd = [None] * NBUF
            sd = [None] * NBUF
            for jl in range(SB):
                b = jl % NBUF
                if sd[b] is not None:
                    sd[b].wait()
                    sd[b] = None
                gd[b] = pltpu.async_copy(src_sp.at[idxc.at[cur, jl]],
                                         ms[b], gsems[b])
                jp = jl - (NBUF - 1)
                if jp >= 0:
                    bp = jp % NBUF
                    gd[bp].wait()
                    sd[bp] = pltpu.async_copy(
                        ms[bp], g_sp.at[idxr.at[cur, jp]], ssems[bp], add=True)
            for jp in range(SB - NBUF + 1, SB):
                bp = jp % NBUF
                gd[bp].wait()
                sd[bp] = pltpu.async_copy(
                    ms[bp], g_sp.at[idxr.at[cur, jp]], ssems[bp], add=True)
            for b in range(NBUF):
                if sd[b] is not None:
                    sd[b].wait()
            if u + 1 < NSUP:
                pr.wait()
                pc.wait()
        plsc.subcore_barrier()

        def scale_chunk(c, _):
            pltpu.sync_copy(g_sp.at[pl.ds(rbase + c * SCCH, SCCH), :],
                            m0.at[pl.ds(0, SCCH), :])

            def grp(g, _):
                sv16 = sbuf[pl.ds(c * SCCH + g * 16, 16)]
                dv16 = sv16 * sv16
                for r16 in range(16):
                    r = g * 16 + r16
                    dv = dv16[r16]
                    for d in range(DH // 16):
                        sl = pl.ds(d * 16, 16)
                        m0[r, sl] = m0[r, sl] * dv
                return 0

            lax.fori_loop(0, SCCH // 16, grp, 0)
            pltpu.sync_copy(m0.at[pl.ds(0, SCCH), :],
                            f_out.at[pl.ds(jbase + c * SCCH, SCCH), :])
            return 0

        lax.fori_loop(0, NSC, scale_chunk, 0)
        plsc.subcore_barrier()


@functools.partial(
    pl.kernel,
    out_type=jax.ShapeDtypeStruct((B,), F32),
    mesh=MESH,
    compiler_params=CPARAMS,
    scratch_types=[
        pltpu.VMEM((PPT,), I32),
        pltpu.VMEM((PPT,), I32),
        pltpu.VMEM((PPT, DH), F32),
        pltpu.VMEM((PPT, DH), F32),
        pltpu.VMEM((PPT, DH), F32),
        pltpu.VMEM((PPT, DH), F32),
        pltpu.VMEM((PPT,), F32),
        pltpu.VMEM((PPT,), F32),
        pltpu.VMEM((PPT,), F32),
        pltpu.SemaphoreType.DMA,
        pltpu.SemaphoreType.DMA,
        pltpu.SemaphoreType.DMA,
        pltpu.SemaphoreType.DMA,
    ],
)
def _k_decode(f0lo, f0hi, f1lo, f1hi, f2lo, f2hi, f3lo, f3hi,
              s_flat, uj_hbm, ij_hbm, gamma,
              uidx, iidx, bul, buh, bil, bih, su, si, gbuf, d0, d1, d2, d3):
    cid = lax.axis_index("c")
    sid = lax.axis_index("s")
    base = (cid * NS + sid) * PPT
    pltpu.sync_copy(uj_hbm.at[pl.ds(base, PPT)], uidx)
    pltpu.sync_copy(ij_hbm.at[pl.ds(base, PPT)], iidx)
    chains = (
        ((f0lo, bul, uidx, d0), (f1lo, bul, uidx, d0),
         (f2lo, bul, uidx, d0), (f3lo, bul, uidx, d0)),
        ((f0hi, buh, uidx, d1), (f1hi, buh, uidx, d1),
         (f2hi, buh, uidx, d1), (f3hi, buh, uidx, d1)),
        ((f0lo, bil, iidx, d2), (f1lo, bil, iidx, d2),
         (f2lo, bil, iidx, d2), (f3lo, bil, iidx, d2)),
        ((f0hi, bih, iidx, d3), (f1hi, bih, iidx, d3),
         (f2hi, bih, iidx, d3), (f3hi, bih, iidx, d3)),
    )
    prev = [None] * 4
    for stage in range(4):
        for ci, chain in enumerate(chains):
            tab, buf, idx, sem = chain[stage]
            if prev[ci] is not None:
                prev[ci].wait()
            prev[ci] = pltpu.async_copy(tab.at[idx], buf, sem,
                                        add=(stage > 0))
    pltpu.sync_copy(s_flat.at[uidx], su)
    pltpu.sync_copy(s_flat.at[iidx], si)
    for dsc in prev:
        dsc.wait()

    def grp(g, _):
        rows = lax.iota(I32, 16) + g * 16
        acc = jnp.zeros((16,), F32)
        for d in range(DH):
            cols = jnp.full((16,), d, I32)
            acc = acc + (plsc.load_gather(bul, [rows, cols])
                         * plsc.load_gather(bil, [rows, cols]))
            acc = acc + (plsc.load_gather(buh, [rows, cols])
                         * plsc.load_gather(bih, [rows, cols]))
        sl = pl.ds(g * 16, 16)
        gbuf[sl] = acc / (su[sl] * si[sl] * 16.0)
        return 0

    lax.fori_loop(0, PPT // 16, grp, 0)
    pltpu.sync_copy(gbuf, gamma.at[pl.ds(base, PPT)])


@jax.jit
def kernel(user_emb, item_emb, edge_index, users, items):
    src = edge_index[0].astype(I32)
    dst = edge_index[1].astype(I32)
    padr = jnp.full((EPAD - E,), JUNK, I32)
    padc = jnp.zeros((EPAD - E,), I32)
    rows3d = jnp.stack([
        jnp.concatenate([src, padr]),
        jnp.concatenate([dst, padr]),
    ]).reshape(NC, NS * CPT, CH)
    cols3d = jnp.stack([
        jnp.concatenate([dst, padc]),
        jnp.concatenate([src, padc]),
    ]).reshape(NC, NS * CPT, CH)
    zpad = jnp.zeros((NPAD - NU, D), F32)
    e0p = jnp.concatenate([user_emb, zpad, item_emb, zpad], axis=0)

    s1d, f0lo, f0hi = _k_deg(rows3d, e0p)
    f1lo, f1hi = _k_layer(rows3d, cols3d, f0lo, f0hi, s1d)
    f2lo, f2hi = _k_layer(rows3d, cols3d, f1lo, f1hi, s1d)
    f3lo, f3hi = _k_layer(rows3d, cols3d, f2lo, f2hi, s1d)
    gamma = _k_decode(f0lo, f0hi, f1lo, f1hi, f2lo, f2hi, f3lo, f3hi,
                      s1d, users.astype(I32), (items.astype(I32) + NPAD))
    return gamma
